# Initial kernel scaffold; baseline (speedup 1.0000x reference)
import jax
import jax.numpy as jnp
from jax.experimental import pallas as pl


def kernel(support_sets, proto_embs):
    raise NotImplementedError("write your pallas kernel here")


# speedup vs baseline: 21.1088x; 21.1088x over previous
import functools

import jax
import jax.numpy as jnp
from jax.experimental import pallas as pl
from jax.experimental.pallas import tpu as pltpu

B, S, D, P, TOPK = 1024, 50, 128, 1024, 20
NEG_BIG = -3.0e38
MASK_FILL = -1.0e7


def _fused_kernel(ss_ref, proto_ref, bool_ref, emb_ref, *, bb):
    ss = ss_ref[...].reshape(bb * S, D)
    proto = proto_ref[...]
    sim = jax.lax.dot_general(
        ss, proto,
        dimension_numbers=(((1,), (1,)), ((), ())),
        preferred_element_type=jnp.float32,
    )
    work = sim
    thresh = None
    for _ in range(TOPK):
        thresh = jnp.max(work, axis=-1, keepdims=True)
        work = jnp.where(work == thresh, NEG_BIG, work)
    sim3 = sim.reshape(bb, S, P)
    t3 = thresh.reshape(bb, S, 1)
    in_topk = (sim3 >= t3).astype(jnp.float32)
    cnt = jnp.sum(in_topk, axis=1)
    mask = cnt >= jnp.float32(S)
    mean = jnp.mean(sim3, axis=1)
    masked = jnp.where(mask, mean, jnp.float32(MASK_FILL))
    m = jnp.max(masked, axis=-1, keepdims=True)
    e = jnp.exp(masked - m)
    dist = e / jnp.sum(e, axis=-1, keepdims=True)
    emb = jax.lax.dot_general(
        dist, proto,
        dimension_numbers=(((1,), (0,)), ((), ())),
        preferred_element_type=jnp.float32,
    )
    norm = jnp.sqrt(jnp.sum(emb * emb, axis=-1, keepdims=True))
    emb = emb / jnp.maximum(norm, jnp.float32(1e-12))
    bool_ref[...] = mask
    emb_ref[...] = emb


def kernel(support_sets, proto_embs):
    bb = 16
    grid = (B // bb,)
    f = functools.partial(_fused_kernel, bb=bb)
    out_bool, out_emb = pl.pallas_call(
        f,
        grid=grid,
        in_specs=[
            pl.BlockSpec((bb, S, D), lambda i: (i, 0, 0)),
            pl.BlockSpec((P, D), lambda i: (0, 0)),
        ],
        out_specs=[
            pl.BlockSpec((bb, P), lambda i: (i, 0)),
            pl.BlockSpec((bb, D), lambda i: (i, 0)),
        ],
        out_shape=[
            jax.ShapeDtypeStruct((B, P), jnp.bool_),
            jax.ShapeDtypeStruct((B, D), jnp.float32),
        ],
    )(support_sets, proto_embs)
    return out_bool, out_emb


# speedup vs baseline: 22.8700x; 1.0834x over previous
import functools

import jax
import jax.numpy as jnp
from jax.experimental import pallas as pl
from jax.experimental.pallas import tpu as pltpu

B, S, D, P, TOPK = 1024, 50, 128, 1024, 20
NEG_BIG = -3.0e38
MASK_FILL = -1.0e7


_SORT8 = [
    (0, 1), (2, 3), (4, 5), (6, 7),
    (0, 2), (1, 3), (4, 6), (5, 7),
    (1, 2), (5, 6),
    (0, 4), (1, 5), (2, 6), (3, 7),
    (2, 4), (3, 5),
    (1, 2), (3, 4), (5, 6),
]


def _fused_kernel(ss_ref, proto_ref, bool_ref, emb_ref, *, bb):
    ss = ss_ref[...].reshape(bb * S, D)
    proto = proto_ref[...]
    sim = jax.lax.dot_general(
        ss, proto,
        dimension_numbers=(((1,), (1,)), ((), ())),
        preferred_element_type=jnp.float32,
    )
    lvl = [sim[:, 128 * j:128 * (j + 1)] for j in range(8)]
    for i, j in _SORT8:
        hi = jnp.maximum(lvl[i], lvl[j])
        lo = jnp.minimum(lvl[i], lvl[j])
        lvl[i], lvl[j] = hi, lo
    thresh = None
    for it in range(TOPK):
        thresh = jnp.max(lvl[0], axis=-1, keepdims=True)
        if it < TOPK - 1:
            popm = lvl[0] == thresh
            for j in range(7):
                lvl[j] = jnp.where(popm, lvl[j + 1], lvl[j])
            lvl[7] = jnp.where(popm, NEG_BIG, lvl[7])
    sim3 = sim.reshape(bb, S, P)
    t3 = thresh.reshape(bb, S, 1)
    in_topk = (sim3 >= t3).astype(jnp.float32)
    cnt = jnp.sum(in_topk, axis=1)
    mask = cnt >= jnp.float32(S)
    mean = jnp.mean(sim3, axis=1)
    masked = jnp.where(mask, mean, jnp.float32(MASK_FILL))
    m = jnp.max(masked, axis=-1, keepdims=True)
    e = jnp.exp(masked - m)
    dist = e / jnp.sum(e, axis=-1, keepdims=True)
    emb = jax.lax.dot_general(
        dist, proto,
        dimension_numbers=(((1,), (0,)), ((), ())),
        preferred_element_type=jnp.float32,
    )
    norm = jnp.sqrt(jnp.sum(emb * emb, axis=-1, keepdims=True))
    emb = emb / jnp.maximum(norm, jnp.float32(1e-12))
    bool_ref[...] = mask
    emb_ref[...] = emb


def kernel(support_sets, proto_embs):
    bb = 16
    grid = (B // bb,)
    f = functools.partial(_fused_kernel, bb=bb)
    out_bool, out_emb = pl.pallas_call(
        f,
        grid=grid,
        in_specs=[
            pl.BlockSpec((bb, S, D), lambda i: (i, 0, 0)),
            pl.BlockSpec((P, D), lambda i: (0, 0)),
        ],
        out_specs=[
            pl.BlockSpec((bb, P), lambda i: (i, 0)),
            pl.BlockSpec((bb, D), lambda i: (i, 0)),
        ],
        out_shape=[
            jax.ShapeDtypeStruct((B, P), jnp.bool_),
            jax.ShapeDtypeStruct((B, D), jnp.float32),
        ],
    )(support_sets, proto_embs)
    return out_bool, out_emb
